# Initial kernel scaffold; baseline (speedup 1.0000x reference)
import jax
import jax.numpy as jnp
from jax.experimental import pallas as pl


def kernel(x, edge_index, edge_attr, u, We, be, Wp, bp, Wn, bn, Wpe, bpe, Wpn, bpn, Wg, bg):
    raise NotImplementedError("write your pallas kernel here")


# speedup vs baseline: 2.8068x; 2.8068x over previous
import functools

import jax
import jax.numpy as jnp
from jax import lax
from jax.experimental import pallas as pl
from jax.experimental.pallas import tpu as pltpu
from jax.experimental.pallas import tpu_sc as plsc

F32 = jnp.float32

_NC = 2
_NS = 16
_NW = _NC * _NS


def _precompute(x, Wrs, u2, Weu, be2, nblk):
    n, d = x.shape
    deo = Weu.shape[1]
    blk = n // nblk

    def body(x_ref, w_ref, u_ref, weu_ref, be_ref, xr_ref, xs_ref, cv_ref):
        t = jnp.dot(x_ref[...], w_ref[...], preferred_element_type=F32)
        xr_ref[...] = t[:, :deo]
        xs_ref[...] = t[:, deo:]

        @pl.when(pl.program_id(0) == 0)
        def _():
            cv_ref[...] = (
                jnp.dot(u_ref[...], weu_ref[...], preferred_element_type=F32)
                + be_ref[...]
            )

    return pl.pallas_call(
        body,
        grid=(nblk,),
        in_specs=[
            pl.BlockSpec((blk, d), lambda i: (i, 0)),
            pl.BlockSpec((d, 2 * deo), lambda i: (0, 0)),
            pl.BlockSpec(u2.shape, lambda i: (0, 0)),
            pl.BlockSpec(Weu.shape, lambda i: (0, 0)),
            pl.BlockSpec(be2.shape, lambda i: (0, 0)),
        ],
        out_specs=[
            pl.BlockSpec((blk, deo), lambda i: (i, 0)),
            pl.BlockSpec((blk, deo), lambda i: (i, 0)),
            pl.BlockSpec((1, deo), lambda i: (0, 0)),
        ],
        out_shape=[
            jax.ShapeDtypeStruct((n, deo), F32),
            jax.ShapeDtypeStruct((n, deo), F32),
            jax.ShapeDtypeStruct((1, deo), F32),
        ],
    )(x, Wrs, u2, Weu, be2)


def _sc_gather(dst, src, xr, xs, E, DEO, CH, NB):
    epw = E // _NW
    m = epw // CH
    nouter = m // NB

    mesh = plsc.VectorSubcoreMesh(core_axis_name="c", subcore_axis_name="s")
    scratch = (
        [pltpu.VMEM((CH,), jnp.int32) for _ in range(2 * NB)]
        + [pltpu.VMEM((CH, DEO), F32) for _ in range(2 * NB)]
        + [pltpu.SemaphoreType.DMA((3 * NB,))]
    )

    @functools.partial(
        pl.kernel,
        out_type=(
            jax.ShapeDtypeStruct((E, DEO), F32),
            jax.ShapeDtypeStruct((E, DEO), F32),
        ),
        mesh=mesh,
        scratch_types=scratch,
    )
    def k(dst_h, src_h, xr_h, xs_h, gd_h, gs_h, *sc):
        idx_d = sc[0:NB]
        idx_s = sc[NB : 2 * NB]
        row_d = sc[2 * NB : 3 * NB]
        row_s = sc[3 * NB : 4 * NB]
        sems = sc[4 * NB]
        wid = lax.axis_index("s") * _NC + lax.axis_index("c")
        base = wid * epw

        def outer(t, carry):
            cps = []
            for r in range(NB):
                off = base + (t * NB + r) * CH
                cps.append(pltpu.async_copy(
                    dst_h.at[pl.ds(off, CH)], idx_d[r], sems.at[r]))
                cps.append(pltpu.async_copy(
                    src_h.at[pl.ds(off, CH)], idx_s[r], sems.at[r]))
            for c in cps:
                c.wait()
            cps = []
            for r in range(NB):
                cps.append(pltpu.async_copy(
                    xr_h.at[idx_d[r]], row_d[r], sems.at[NB + r]))
                cps.append(pltpu.async_copy(
                    xs_h.at[idx_s[r]], row_s[r], sems.at[NB + r]))
            for c in cps:
                c.wait()
            cps = []
            for r in range(NB):
                off = base + (t * NB + r) * CH
                cps.append(pltpu.async_copy(
                    row_d[r], gd_h.at[pl.ds(off, CH)], sems.at[2 * NB + r]))
                cps.append(pltpu.async_copy(
                    row_s[r], gs_h.at[pl.ds(off, CH)], sems.at[2 * NB + r]))
            for c in cps:
                c.wait()
            return carry

        lax.fori_loop(0, nouter, outer, 0)

    return k(dst, src, xr, xs)


def _edge_block(gd, gs, ea, Wee, cvec, Wpq, bpq, eblk):
    E, deo = gd.shape
    de = ea.shape[1]
    dq = Wpq.shape[1]
    nblk = E // eblk

    def body(gd_ref, gs_ref, ea_ref, wee_ref, cv_ref, wpq_ref, bpq_ref,
             en_ref, pj_ref, pe_ref):
        pre = (
            gd_ref[...] + gs_ref[...]
            + jnp.dot(ea_ref[...], wee_ref[...], preferred_element_type=F32)
            + cv_ref[...]
        )
        e_new = jnp.maximum(pre, 0.0)
        en_ref[...] = e_new
        pq = jnp.maximum(
            jnp.dot(e_new, wpq_ref[...], preferred_element_type=F32)
            + bpq_ref[...],
            0.0,
        )
        h = deo // 2
        pj_ref[...] = jnp.stack([pq[:, :h], pq[:, h : 2 * h]], axis=0)
        part = jnp.sum(pq[:, deo:], axis=0, keepdims=True)

        @pl.when(pl.program_id(0) == 0)
        def _():
            pe_ref[...] = jnp.zeros_like(pe_ref)

        pe_ref[...] += part

    return pl.pallas_call(
        body,
        grid=(nblk,),
        in_specs=[
            pl.BlockSpec((eblk, deo), lambda i: (i, 0)),
            pl.BlockSpec((eblk, deo), lambda i: (i, 0)),
            pl.BlockSpec((eblk, de), lambda i: (i, 0)),
            pl.BlockSpec(Wee.shape, lambda i: (0, 0)),
            pl.BlockSpec(cvec.shape, lambda i: (0, 0)),
            pl.BlockSpec(Wpq.shape, lambda i: (0, 0)),
            pl.BlockSpec(bpq.shape, lambda i: (0, 0)),
        ],
        out_specs=[
            pl.BlockSpec((eblk, deo), lambda i: (i, 0)),
            pl.BlockSpec((2, eblk, deo // 2), lambda i: (0, i, 0)),
            pl.BlockSpec((1, deo), lambda i: (0, 0)),
        ],
        out_shape=[
            jax.ShapeDtypeStruct((E, deo), F32),
            jax.ShapeDtypeStruct((2, E, deo // 2), F32),
            jax.ShapeDtypeStruct((1, deo), F32),
        ],
    )(gd, gs, ea, Wee, cvec, Wpq, bpq)


def _sc_scatter(dst, proj2, z2d, ones_in, N, E, CH, NB):
    h = proj2.shape[2]
    epw = E // _NS
    nouter = (epw // CH) // NB
    epw_c = E // (2 * _NS)
    nouter_c = (epw_c // CH) // NB
    rpt = (N // _NS) // 8 * 8
    rlast = N - (_NS - 1) * rpt

    mesh = plsc.VectorSubcoreMesh(core_axis_name="c", subcore_axis_name="s")
    scratch = (
        [pltpu.VMEM((CH,), jnp.int32) for _ in range(NB)]
        + [pltpu.VMEM((CH, h), F32) for _ in range(NB)]
        + [
            pltpu.VMEM((CH, h), F32),
            pltpu.VMEM_SHARED((N, h), F32),
            pltpu.SemaphoreType.DMA((2 * NB,)),
        ]
    )

    @functools.partial(
        pl.kernel,
        out_type=(
            jax.ShapeDtypeStruct((2, N, h), F32),
            jax.ShapeDtypeStruct((2, N, h), F32),
        ),
        mesh=mesh,
        scratch_types=scratch,
    )
    def k(dst_h, pj_h, z2_h, ones_h, sums_h, cnt_h, *sc):
        idx = sc[0:NB]
        pbuf = sc[NB : 2 * NB]
        ones = sc[2 * NB]
        acc = sc[2 * NB + 1]
        sems = sc[2 * NB + 2]
        cid = lax.axis_index("c")
        sid = lax.axis_index("s")

        def zero_acc():
            @pl.when(sid < _NS - 1)
            def _():
                pltpu.sync_copy(z2_h.at[pl.ds(sid * rpt, rpt)],
                                acc.at[pl.ds(sid * rpt, rpt)])

            @pl.when(sid == _NS - 1)
            def _():
                pltpu.sync_copy(z2_h.at[pl.ds((_NS - 1) * rpt, rlast)],
                                acc.at[pl.ds((_NS - 1) * rpt, rlast)])

        def write_acc(out3d):
            @pl.when(sid < _NS - 1)
            def _():
                pltpu.sync_copy(acc.at[pl.ds(sid * rpt, rpt)],
                                out3d.at[cid, pl.ds(sid * rpt, rpt)])

            @pl.when(sid == _NS - 1)
            def _():
                pltpu.sync_copy(acc.at[pl.ds((_NS - 1) * rpt, rlast)],
                                out3d.at[cid, pl.ds((_NS - 1) * rpt, rlast)])

        zero_acc()
        pltpu.sync_copy(ones_h, ones)
        plsc.subcore_barrier()

        base = sid * epw

        def outer(t, carry):
            cps = []
            for r in range(NB):
                off = base + (t * NB + r) * CH
                cps.append(pltpu.async_copy(
                    dst_h.at[pl.ds(off, CH)], idx[r], sems.at[r]))
                cps.append(pltpu.async_copy(
                    pj_h.at[cid, pl.ds(off, CH)], pbuf[r], sems.at[NB + r]))
            for c in cps:
                c.wait()
            for r in range(NB):
                pltpu.sync_copy(pbuf[r], acc.at[idx[r]], add=True)
            return carry

        lax.fori_loop(0, nouter, outer, 0)
        plsc.subcore_barrier()
        write_acc(sums_h)
        plsc.subcore_barrier()

        zero_acc()
        plsc.subcore_barrier()

        base_c = cid * (E // 2) + sid * epw_c

        def outer_c(t, carry):
            cps = []
            for r in range(NB):
                off = base_c + (t * NB + r) * CH
                cps.append(pltpu.async_copy(
                    dst_h.at[pl.ds(off, CH)], idx[r], sems.at[r]))
            for c in cps:
                c.wait()
            for r in range(NB):
                pltpu.sync_copy(ones, acc.at[idx[r]], add=True)
            return carry

        lax.fori_loop(0, nouter_c, outer_c, 0)
        plsc.subcore_barrier()
        write_acc(cnt_h)

    return k(dst, proj2, z2d, ones_in)


def _node_global(sums2, cnt2, Wn, bn2, Wpn, bpn2, pe_sum, u2, Wg, bg2,
                 E, nblk):
    N = sums2.shape[1]
    h = sums2.shape[2]
    d = Wn.shape[0]
    dg = u2.shape[1]
    blk = N // nblk

    def body(s_ref, c_ref, wn_ref, bn_ref, wpn_ref, bpn_ref, pe_ref,
             u_ref, wg_ref, bg_ref, n_ref, u_out_ref, acc):
        s = jnp.concatenate([s_ref[0], s_ref[1]], axis=1)
        cnt = c_ref[0][:, 0:1] + c_ref[1][:, 0:1]
        agg = s / jnp.maximum(cnt, 1.0)
        nb = jnp.maximum(
            jnp.dot(agg, wn_ref[...], preferred_element_type=F32)
            + bn_ref[...],
            0.0,
        )
        n_ref[...] = nb
        part = jnp.sum(
            jnp.maximum(
                jnp.dot(nb, wpn_ref[...], preferred_element_type=F32)
                + bpn_ref[...],
                0.0,
            ),
            axis=0,
            keepdims=True,
        )

        @pl.when(pl.program_id(0) == 0)
        def _():
            acc[...] = jnp.zeros_like(acc)

        acc[...] += part

        @pl.when(pl.program_id(0) == pl.num_programs(0) - 1)
        def _():
            ge = pe_ref[...] / float(E)
            gn = acc[...] / float(N)
            g = (
                jnp.dot(ge, wg_ref[: 2 * h, :], preferred_element_type=F32)
                + jnp.dot(gn, wg_ref[2 * h : 2 * h + d, :],
                          preferred_element_type=F32)
                + jnp.dot(u_ref[...], wg_ref[2 * h + d :, :],
                          preferred_element_type=F32)
                + bg_ref[...]
            )
            u_out_ref[...] = jnp.maximum(g, 0.0)

    return pl.pallas_call(
        body,
        grid=(nblk,),
        in_specs=[
            pl.BlockSpec((2, blk, h), lambda i: (0, i, 0)),
            pl.BlockSpec((2, blk, h), lambda i: (0, i, 0)),
            pl.BlockSpec(Wn.shape, lambda i: (0, 0)),
            pl.BlockSpec(bn2.shape, lambda i: (0, 0)),
            pl.BlockSpec(Wpn.shape, lambda i: (0, 0)),
            pl.BlockSpec(bpn2.shape, lambda i: (0, 0)),
            pl.BlockSpec(pe_sum.shape, lambda i: (0, 0)),
            pl.BlockSpec(u2.shape, lambda i: (0, 0)),
            pl.BlockSpec(Wg.shape, lambda i: (0, 0)),
            pl.BlockSpec(bg2.shape, lambda i: (0, 0)),
        ],
        out_specs=[
            pl.BlockSpec((blk, d), lambda i: (i, 0)),
            pl.BlockSpec((1, dg), lambda i: (0, 0)),
        ],
        out_shape=[
            jax.ShapeDtypeStruct((N, d), F32),
            jax.ShapeDtypeStruct((1, dg), F32),
        ],
        scratch_shapes=[pltpu.VMEM((1, d), F32)],
    )(sums2, cnt2, Wn, bn2, Wpn, bpn2, pe_sum, u2, Wg, bg2)


def kernel(x, edge_index, edge_attr, u, We, be, Wp, bp, Wn, bn,
           Wpe, bpe, Wpn, bpn, Wg, bg):
    N, D = x.shape
    E = edge_attr.shape[0]
    DE = edge_attr.shape[1]
    DG = u.shape[0]
    DEO = We.shape[1]

    src = edge_index[0]
    dst = edge_index[1]

    Wee = We[:DE]
    Wrs = jnp.concatenate([We[DE : DE + D], We[DE + D : DE + 2 * D]], axis=1)
    Weu = We[DE + 2 * D :]
    u2 = u.reshape(1, DG)
    be2 = be.reshape(1, DEO)
    Wpq = jnp.concatenate([Wp, Wpe], axis=1)
    bpq = jnp.concatenate([bp, bpe]).reshape(1, 2 * DEO)

    xr, xs, cvec = _precompute(x, Wrs, u2, Weu, be2, nblk=10)
    gd, gs = _sc_gather(dst, src, xr, xs, E, DEO, CH=40, NB=5)
    e_new, proj2, pe_sum = _edge_block(gd, gs, edge_attr, Wee, cvec,
                                       Wpq, bpq, eblk=2000)
    z2d = jnp.zeros((N, DEO // 2), F32)
    ones_in = jnp.ones((40, DEO // 2), F32)
    sums2, cnt2 = _sc_scatter(dst, proj2, z2d, ones_in, N, E, CH=40, NB=5)
    n_new, u_new2 = _node_global(sums2, cnt2, Wn, bn.reshape(1, D),
                                 Wpn, bpn.reshape(1, D), pe_sum, u2,
                                 Wg, bg.reshape(1, DG), E, nblk=10)
    return e_new, n_new, u_new2.reshape(DG)


# speedup vs baseline: 3.0013x; 1.0693x over previous
import functools

import jax
import jax.numpy as jnp
from jax import lax
from jax.experimental import pallas as pl
from jax.experimental.pallas import tpu as pltpu
from jax.experimental.pallas import tpu_sc as plsc

F32 = jnp.float32

_NC = 2
_NS = 16
_NW = _NC * _NS


def _precompute(x, Wrs, u2, Weu, be2, nblk):
    n, d = x.shape
    deo = Weu.shape[1]
    blk = n // nblk

    def body(x_ref, w_ref, u_ref, weu_ref, be_ref, xr_ref, xs_ref, cv_ref):
        t = jnp.dot(x_ref[...], w_ref[...], preferred_element_type=F32)
        xr_ref[...] = t[:, :deo]
        xs_ref[...] = t[:, deo:]

        @pl.when(pl.program_id(0) == 0)
        def _():
            cv_ref[...] = (
                jnp.dot(u_ref[...], weu_ref[...], preferred_element_type=F32)
                + be_ref[...]
            )

    return pl.pallas_call(
        body,
        grid=(nblk,),
        in_specs=[
            pl.BlockSpec((blk, d), lambda i: (i, 0)),
            pl.BlockSpec((d, 2 * deo), lambda i: (0, 0)),
            pl.BlockSpec(u2.shape, lambda i: (0, 0)),
            pl.BlockSpec(Weu.shape, lambda i: (0, 0)),
            pl.BlockSpec(be2.shape, lambda i: (0, 0)),
        ],
        out_specs=[
            pl.BlockSpec((blk, deo), lambda i: (i, 0)),
            pl.BlockSpec((blk, deo), lambda i: (i, 0)),
            pl.BlockSpec((1, deo), lambda i: (0, 0)),
        ],
        out_shape=[
            jax.ShapeDtypeStruct((n, deo), F32),
            jax.ShapeDtypeStruct((n, deo), F32),
            jax.ShapeDtypeStruct((1, deo), F32),
        ],
    )(x, Wrs, u2, Weu, be2)


def _sc_gather(dst, src, xr, xs, E, DEO, CH, NB):
    epw = E // _NW
    m = epw // CH
    nouter = m // NB

    mesh = plsc.VectorSubcoreMesh(core_axis_name="c", subcore_axis_name="s")
    scratch = (
        [pltpu.VMEM((CH,), jnp.int32) for _ in range(2 * NB)]
        + [pltpu.VMEM((CH, DEO), F32) for _ in range(2 * NB)]
        + [pltpu.SemaphoreType.DMA((3 * NB,))]
    )

    @functools.partial(
        pl.kernel,
        out_type=(
            jax.ShapeDtypeStruct((E, DEO), F32),
            jax.ShapeDtypeStruct((E, DEO), F32),
        ),
        mesh=mesh,
        scratch_types=scratch,
    )
    def k(dst_h, src_h, xr_h, xs_h, gd_h, gs_h, *sc):
        idx_d = sc[0:NB]
        idx_s = sc[NB : 2 * NB]
        row_d = sc[2 * NB : 3 * NB]
        row_s = sc[3 * NB : 4 * NB]
        sems = sc[4 * NB]
        wid = lax.axis_index("s") * _NC + lax.axis_index("c")
        base = wid * epw

        def outer(t, carry):
            cps = []
            for r in range(NB):
                off = base + (t * NB + r) * CH
                cps.append(pltpu.async_copy(
                    dst_h.at[pl.ds(off, CH)], idx_d[r], sems.at[r]))
                cps.append(pltpu.async_copy(
                    src_h.at[pl.ds(off, CH)], idx_s[r], sems.at[r]))
            for c in cps:
                c.wait()
            cps = []
            for r in range(NB):
                cps.append(pltpu.async_copy(
                    xr_h.at[idx_d[r]], row_d[r], sems.at[NB + r]))
                cps.append(pltpu.async_copy(
                    xs_h.at[idx_s[r]], row_s[r], sems.at[NB + r]))
            for c in cps:
                c.wait()
            cps = []
            for r in range(NB):
                off = base + (t * NB + r) * CH
                cps.append(pltpu.async_copy(
                    row_d[r], gd_h.at[pl.ds(off, CH)], sems.at[2 * NB + r]))
                cps.append(pltpu.async_copy(
                    row_s[r], gs_h.at[pl.ds(off, CH)], sems.at[2 * NB + r]))
            for c in cps:
                c.wait()
            return carry

        lax.fori_loop(0, nouter, outer, 0)

    return k(dst, src, xr, xs)


def _edge_block(gd, gs, ea, Wee, cvec, Wpq, bpq, eblk):
    E, deo = gd.shape
    de = ea.shape[1]
    dq = Wpq.shape[1]
    nblk = E // eblk

    def body(gd_ref, gs_ref, ea_ref, wee_ref, cv_ref, wpq_ref, bpq_ref,
             en_ref, pj_ref, pe_ref):
        pre = (
            gd_ref[...] + gs_ref[...]
            + jnp.dot(ea_ref[...], wee_ref[...], preferred_element_type=F32)
            + cv_ref[...]
        )
        e_new = jnp.maximum(pre, 0.0)
        en_ref[...] = e_new
        pq = jnp.maximum(
            jnp.dot(e_new.astype(jnp.bfloat16),
                    wpq_ref[...].astype(jnp.bfloat16),
                    preferred_element_type=F32)
            + bpq_ref[...],
            0.0,
        )
        h = deo // 2
        pj_ref[...] = jnp.stack([pq[:, :h], pq[:, h : 2 * h]], axis=0)
        part = jnp.sum(pq[:, deo:], axis=0, keepdims=True)

        @pl.when(pl.program_id(0) == 0)
        def _():
            pe_ref[...] = jnp.zeros_like(pe_ref)

        pe_ref[...] += part

    return pl.pallas_call(
        body,
        grid=(nblk,),
        in_specs=[
            pl.BlockSpec((eblk, deo), lambda i: (i, 0)),
            pl.BlockSpec((eblk, deo), lambda i: (i, 0)),
            pl.BlockSpec((eblk, de), lambda i: (i, 0)),
            pl.BlockSpec(Wee.shape, lambda i: (0, 0)),
            pl.BlockSpec(cvec.shape, lambda i: (0, 0)),
            pl.BlockSpec(Wpq.shape, lambda i: (0, 0)),
            pl.BlockSpec(bpq.shape, lambda i: (0, 0)),
        ],
        out_specs=[
            pl.BlockSpec((eblk, deo), lambda i: (i, 0)),
            pl.BlockSpec((2, eblk, deo // 2), lambda i: (0, i, 0)),
            pl.BlockSpec((1, deo), lambda i: (0, 0)),
        ],
        out_shape=[
            jax.ShapeDtypeStruct((E, deo), F32),
            jax.ShapeDtypeStruct((2, E, deo // 2), F32),
            jax.ShapeDtypeStruct((1, deo), F32),
        ],
    )(gd, gs, ea, Wee, cvec, Wpq, bpq)


def _sc_scatter(dst, proj2, z2d, N, E, CH, NB):
    h = proj2.shape[2]
    epw = E // _NS
    nouter = (epw // CH) // NB
    rpt = (N // _NS) // 8 * 8
    rlast = N - (_NS - 1) * rpt

    mesh = plsc.VectorSubcoreMesh(core_axis_name="c", subcore_axis_name="s")
    scratch = (
        [pltpu.VMEM((CH,), jnp.int32) for _ in range(NB)]
        + [pltpu.VMEM((CH, h), F32) for _ in range(NB)]
        + [
            pltpu.VMEM_SHARED((N, h), F32),
            pltpu.SemaphoreType.DMA((2 * NB,)),
        ]
    )

    @functools.partial(
        pl.kernel,
        out_type=jax.ShapeDtypeStruct((2, N, h), F32),
        mesh=mesh,
        scratch_types=scratch,
    )
    def k(dst_h, pj_h, z2_h, sums_h, *sc):
        idx = sc[0:NB]
        pbuf = sc[NB : 2 * NB]
        acc = sc[2 * NB]
        sems = sc[2 * NB + 1]
        cid = lax.axis_index("c")
        sid = lax.axis_index("s")

        def zero_acc():
            @pl.when(sid < _NS - 1)
            def _():
                pltpu.sync_copy(z2_h.at[pl.ds(sid * rpt, rpt)],
                                acc.at[pl.ds(sid * rpt, rpt)])

            @pl.when(sid == _NS - 1)
            def _():
                pltpu.sync_copy(z2_h.at[pl.ds((_NS - 1) * rpt, rlast)],
                                acc.at[pl.ds((_NS - 1) * rpt, rlast)])

        def write_acc(out3d):
            @pl.when(sid < _NS - 1)
            def _():
                pltpu.sync_copy(acc.at[pl.ds(sid * rpt, rpt)],
                                out3d.at[cid, pl.ds(sid * rpt, rpt)])

            @pl.when(sid == _NS - 1)
            def _():
                pltpu.sync_copy(acc.at[pl.ds((_NS - 1) * rpt, rlast)],
                                out3d.at[cid, pl.ds((_NS - 1) * rpt, rlast)])

        zero_acc()
        plsc.subcore_barrier()

        base = sid * epw

        def outer(t, carry):
            cps = []
            for r in range(NB):
                off = base + (t * NB + r) * CH
                cps.append(pltpu.async_copy(
                    dst_h.at[pl.ds(off, CH)], idx[r], sems.at[r]))
                cps.append(pltpu.async_copy(
                    pj_h.at[cid, pl.ds(off, CH)], pbuf[r], sems.at[NB + r]))
            for c in cps:
                c.wait()
            for r in range(NB):
                pltpu.sync_copy(pbuf[r], acc.at[idx[r]], add=True)
            return carry

        lax.fori_loop(0, nouter, outer, 0)
        plsc.subcore_barrier()
        write_acc(sums_h)

    return k(dst, proj2, z2d)


def _sc_counts(dst, z2d, ones_in, N, E, CH, NB):
    h = z2d.shape[1]
    epw_c = E // (2 * _NS)
    nouter_c = (epw_c // CH) // NB
    rpt = (N // _NS) // 8 * 8
    rlast = N - (_NS - 1) * rpt

    mesh = plsc.VectorSubcoreMesh(core_axis_name="c", subcore_axis_name="s")
    scratch = (
        [pltpu.VMEM((CH,), jnp.int32) for _ in range(NB)]
        + [
            pltpu.VMEM((CH, h), F32),
            pltpu.VMEM_SHARED((N, h), F32),
            pltpu.SemaphoreType.DMA((NB,)),
        ]
    )

    @functools.partial(
        pl.kernel,
        out_type=jax.ShapeDtypeStruct((2, N, h), F32),
        mesh=mesh,
        scratch_types=scratch,
    )
    def k(dst_h, z2_h, ones_h, cnt_h, *sc):
        idx = sc[0:NB]
        ones = sc[NB]
        acc = sc[NB + 1]
        sems = sc[NB + 2]
        cid = lax.axis_index("c")
        sid = lax.axis_index("s")

        @pl.when(sid < _NS - 1)
        def _():
            pltpu.sync_copy(z2_h.at[pl.ds(sid * rpt, rpt)],
                            acc.at[pl.ds(sid * rpt, rpt)])

        @pl.when(sid == _NS - 1)
        def _():
            pltpu.sync_copy(z2_h.at[pl.ds((_NS - 1) * rpt, rlast)],
                            acc.at[pl.ds((_NS - 1) * rpt, rlast)])

        pltpu.sync_copy(ones_h, ones)
        plsc.subcore_barrier()

        base_c = cid * (E // 2) + sid * epw_c

        def outer_c(t, carry):
            cps = []
            for r in range(NB):
                off = base_c + (t * NB + r) * CH
                cps.append(pltpu.async_copy(
                    dst_h.at[pl.ds(off, CH)], idx[r], sems.at[r]))
            for c in cps:
                c.wait()
            for r in range(NB):
                pltpu.sync_copy(ones, acc.at[idx[r]], add=True)
            return carry

        lax.fori_loop(0, nouter_c, outer_c, 0)
        plsc.subcore_barrier()

        @pl.when(sid < _NS - 1)
        def _():
            pltpu.sync_copy(acc.at[pl.ds(sid * rpt, rpt)],
                            cnt_h.at[cid, pl.ds(sid * rpt, rpt)])

        @pl.when(sid == _NS - 1)
        def _():
            pltpu.sync_copy(acc.at[pl.ds((_NS - 1) * rpt, rlast)],
                            cnt_h.at[cid, pl.ds((_NS - 1) * rpt, rlast)])

    return k(dst, z2d, ones_in)


def _node_global(sums2, cnt2, Wn, bn2, Wpn, bpn2, pe_sum, u2, Wg, bg2,
                 E, nblk):
    N = sums2.shape[1]
    h = sums2.shape[2]
    d = Wn.shape[0]
    dg = u2.shape[1]
    blk = N // nblk

    def body(s_ref, c_ref, wn_ref, bn_ref, wpn_ref, bpn_ref, pe_ref,
             u_ref, wg_ref, bg_ref, n_ref, u_out_ref, acc):
        s = jnp.concatenate([s_ref[0], s_ref[1]], axis=1)
        cnt = c_ref[0][:, 0:1] + c_ref[1][:, 0:1]
        agg = s / jnp.maximum(cnt, 1.0)
        nb = jnp.maximum(
            jnp.dot(agg, wn_ref[...], preferred_element_type=F32)
            + bn_ref[...],
            0.0,
        )
        n_ref[...] = nb
        part = jnp.sum(
            jnp.maximum(
                jnp.dot(nb, wpn_ref[...], preferred_element_type=F32)
                + bpn_ref[...],
                0.0,
            ),
            axis=0,
            keepdims=True,
        )

        @pl.when(pl.program_id(0) == 0)
        def _():
            acc[...] = jnp.zeros_like(acc)

        acc[...] += part

        @pl.when(pl.program_id(0) == pl.num_programs(0) - 1)
        def _():
            ge = pe_ref[...] / float(E)
            gn = acc[...] / float(N)
            g = (
                jnp.dot(ge, wg_ref[: 2 * h, :], preferred_element_type=F32)
                + jnp.dot(gn, wg_ref[2 * h : 2 * h + d, :],
                          preferred_element_type=F32)
                + jnp.dot(u_ref[...], wg_ref[2 * h + d :, :],
                          preferred_element_type=F32)
                + bg_ref[...]
            )
            u_out_ref[...] = jnp.maximum(g, 0.0)

    return pl.pallas_call(
        body,
        grid=(nblk,),
        in_specs=[
            pl.BlockSpec((2, blk, h), lambda i: (0, i, 0)),
            pl.BlockSpec((2, blk, h), lambda i: (0, i, 0)),
            pl.BlockSpec(Wn.shape, lambda i: (0, 0)),
            pl.BlockSpec(bn2.shape, lambda i: (0, 0)),
            pl.BlockSpec(Wpn.shape, lambda i: (0, 0)),
            pl.BlockSpec(bpn2.shape, lambda i: (0, 0)),
            pl.BlockSpec(pe_sum.shape, lambda i: (0, 0)),
            pl.BlockSpec(u2.shape, lambda i: (0, 0)),
            pl.BlockSpec(Wg.shape, lambda i: (0, 0)),
            pl.BlockSpec(bg2.shape, lambda i: (0, 0)),
        ],
        out_specs=[
            pl.BlockSpec((blk, d), lambda i: (i, 0)),
            pl.BlockSpec((1, dg), lambda i: (0, 0)),
        ],
        out_shape=[
            jax.ShapeDtypeStruct((N, d), F32),
            jax.ShapeDtypeStruct((1, dg), F32),
        ],
        scratch_shapes=[pltpu.VMEM((1, d), F32)],
    )(sums2, cnt2, Wn, bn2, Wpn, bpn2, pe_sum, u2, Wg, bg2)


def kernel(x, edge_index, edge_attr, u, We, be, Wp, bp, Wn, bn,
           Wpe, bpe, Wpn, bpn, Wg, bg):
    N, D = x.shape
    E = edge_attr.shape[0]
    DE = edge_attr.shape[1]
    DG = u.shape[0]
    DEO = We.shape[1]

    src = edge_index[0]
    dst = edge_index[1]

    Wee = We[:DE]
    Wrs = jnp.concatenate([We[DE : DE + D], We[DE + D : DE + 2 * D]], axis=1)
    Weu = We[DE + 2 * D :]
    u2 = u.reshape(1, DG)
    be2 = be.reshape(1, DEO)
    Wpq = jnp.concatenate([Wp, Wpe], axis=1)
    bpq = jnp.concatenate([bp, bpe]).reshape(1, 2 * DEO)

    xr, xs, cvec = _precompute(x, Wrs, u2, Weu, be2, nblk=10)
    gd, gs = _sc_gather(dst, src, xr, xs, E, DEO, CH=40, NB=5)
    e_new, proj2, pe_sum = _edge_block(gd, gs, edge_attr, Wee, cvec,
                                       Wpq, bpq, eblk=2000)
    z2d = jnp.zeros((N, DEO // 2), F32)
    ones_in = jnp.ones((40, DEO // 2), F32)
    cnt2 = _sc_counts(dst, z2d, ones_in, N, E, CH=40, NB=5)
    sums2 = _sc_scatter(dst, proj2, z2d, N, E, CH=40, NB=5)
    n_new, u_new2 = _node_global(sums2, cnt2, Wn, bn.reshape(1, D),
                                 Wpn, bpn.reshape(1, D), pe_sum, u2,
                                 Wg, bg.reshape(1, DG), E, nblk=10)
    return e_new, n_new, u_new2.reshape(DG)


# speedup vs baseline: 3.0932x; 1.0306x over previous
import functools

import jax
import jax.numpy as jnp
from jax import lax
from jax.experimental import pallas as pl
from jax.experimental.pallas import tpu as pltpu
from jax.experimental.pallas import tpu_sc as plsc

F32 = jnp.float32

_NC = 2
_NS = 16
_NW = _NC * _NS


def _precompute(x, Wrs, u2, Weu, be2, nblk):
    n, d = x.shape
    deo = Weu.shape[1]
    blk = n // nblk

    def body(x_ref, w_ref, u_ref, weu_ref, be_ref, xr_ref, xs_ref, cv_ref):
        t = jnp.dot(x_ref[...], w_ref[...], preferred_element_type=F32)
        xr_ref[...] = t[:, :deo]
        xs_ref[...] = t[:, deo:]

        @pl.when(pl.program_id(0) == 0)
        def _():
            cv_ref[...] = (
                jnp.dot(u_ref[...], weu_ref[...], preferred_element_type=F32)
                + be_ref[...]
            )

    return pl.pallas_call(
        body,
        grid=(nblk,),
        in_specs=[
            pl.BlockSpec((blk, d), lambda i: (i, 0)),
            pl.BlockSpec((d, 2 * deo), lambda i: (0, 0)),
            pl.BlockSpec(u2.shape, lambda i: (0, 0)),
            pl.BlockSpec(Weu.shape, lambda i: (0, 0)),
            pl.BlockSpec(be2.shape, lambda i: (0, 0)),
        ],
        out_specs=[
            pl.BlockSpec((blk, deo), lambda i: (i, 0)),
            pl.BlockSpec((blk, deo), lambda i: (i, 0)),
            pl.BlockSpec((1, deo), lambda i: (0, 0)),
        ],
        out_shape=[
            jax.ShapeDtypeStruct((n, deo), F32),
            jax.ShapeDtypeStruct((n, deo), F32),
            jax.ShapeDtypeStruct((1, deo), F32),
        ],
    )(x, Wrs, u2, Weu, be2)


def _sc_gather(dst, src, xr, xs, E, DEO, CH, NB):
    epw = E // _NW
    m = epw // CH
    nouter = m // NB

    mesh = plsc.VectorSubcoreMesh(core_axis_name="c", subcore_axis_name="s")
    scratch = (
        [pltpu.VMEM((CH,), jnp.int32) for _ in range(2 * NB)]
        + [pltpu.VMEM((CH, DEO), F32) for _ in range(2 * NB)]
        + [pltpu.SemaphoreType.DMA((3 * NB,))]
    )

    @functools.partial(
        pl.kernel,
        out_type=(
            jax.ShapeDtypeStruct((E, DEO), F32),
            jax.ShapeDtypeStruct((E, DEO), F32),
        ),
        mesh=mesh,
        scratch_types=scratch,
    )
    def k(dst_h, src_h, xr_h, xs_h, gd_h, gs_h, *sc):
        idx_d = sc[0:NB]
        idx_s = sc[NB : 2 * NB]
        row_d = sc[2 * NB : 3 * NB]
        row_s = sc[3 * NB : 4 * NB]
        sems = sc[4 * NB]
        wid = lax.axis_index("s") * _NC + lax.axis_index("c")
        base = wid * epw

        def wb_addr(tt, r):
            off = base + (tt * NB + r) * CH
            return gd_h.at[pl.ds(off, CH)], gs_h.at[pl.ds(off, CH)]

        for r in range(NB):
            off = base + r * CH
            pltpu.async_copy(dst_h.at[pl.ds(off, CH)], idx_d[r], sems.at[r])
            pltpu.async_copy(src_h.at[pl.ds(off, CH)], idx_s[r], sems.at[r])

        def outer(t, carry):
            @pl.when(t > 0)
            def _():
                for r in range(NB):
                    gd_s, gs_s = wb_addr(t - 1, r)
                    pltpu.make_async_copy(row_d[r], gd_s,
                                          sems.at[2 * NB + r]).wait()
                    pltpu.make_async_copy(row_s[r], gs_s,
                                          sems.at[2 * NB + r]).wait()
            for r in range(NB):
                off = base + (t * NB + r) * CH
                pltpu.make_async_copy(dst_h.at[pl.ds(off, CH)], idx_d[r],
                                      sems.at[r]).wait()
                pltpu.make_async_copy(src_h.at[pl.ds(off, CH)], idx_s[r],
                                      sems.at[r]).wait()
                pltpu.async_copy(xr_h.at[idx_d[r]], row_d[r], sems.at[NB + r])
                pltpu.async_copy(xs_h.at[idx_s[r]], row_s[r], sems.at[NB + r])
            for r in range(NB):
                pltpu.make_async_copy(xr_h.at[idx_d[r]], row_d[r],
                                      sems.at[NB + r]).wait()
                pltpu.make_async_copy(xs_h.at[idx_s[r]], row_s[r],
                                      sems.at[NB + r]).wait()

                @pl.when(t + 1 < nouter)
                def _():
                    off_n = base + ((t + 1) * NB + r) * CH
                    pltpu.async_copy(dst_h.at[pl.ds(off_n, CH)], idx_d[r],
                                     sems.at[r])
                    pltpu.async_copy(src_h.at[pl.ds(off_n, CH)], idx_s[r],
                                     sems.at[r])

                gd_s, gs_s = wb_addr(t, r)
                pltpu.async_copy(row_d[r], gd_s, sems.at[2 * NB + r])
                pltpu.async_copy(row_s[r], gs_s, sems.at[2 * NB + r])
            return carry

        lax.fori_loop(0, nouter, outer, 0)
        for r in range(NB):
            gd_s, gs_s = wb_addr(nouter - 1, r)
            pltpu.make_async_copy(row_d[r], gd_s, sems.at[2 * NB + r]).wait()
            pltpu.make_async_copy(row_s[r], gs_s, sems.at[2 * NB + r]).wait()

    return k(dst, src, xr, xs)


def _edge_block(gd, gs, ea, Wee, cvec, Wpq, bpq, eblk):
    E, deo = gd.shape
    de = ea.shape[1]
    dq = Wpq.shape[1]
    nblk = E // eblk

    def body(gd_ref, gs_ref, ea_ref, wee_ref, cv_ref, wpq_ref, bpq_ref,
             en_ref, pj_ref, pe_ref):
        pre = (
            gd_ref[...] + gs_ref[...]
            + jnp.dot(ea_ref[...], wee_ref[...], preferred_element_type=F32)
            + cv_ref[...]
        )
        e_new = jnp.maximum(pre, 0.0)
        en_ref[...] = e_new
        pq = jnp.maximum(
            jnp.dot(e_new.astype(jnp.bfloat16),
                    wpq_ref[...].astype(jnp.bfloat16),
                    preferred_element_type=F32)
            + bpq_ref[...],
            0.0,
        )
        h = deo // 2
        pj_ref[...] = jnp.stack([pq[:, :h], pq[:, h : 2 * h]], axis=0)
        part = jnp.sum(pq[:, deo:], axis=0, keepdims=True)

        @pl.when(pl.program_id(0) == 0)
        def _():
            pe_ref[...] = jnp.zeros_like(pe_ref)

        pe_ref[...] += part

    return pl.pallas_call(
        body,
        grid=(nblk,),
        in_specs=[
            pl.BlockSpec((eblk, deo), lambda i: (i, 0)),
            pl.BlockSpec((eblk, deo), lambda i: (i, 0)),
            pl.BlockSpec((eblk, de), lambda i: (i, 0)),
            pl.BlockSpec(Wee.shape, lambda i: (0, 0)),
            pl.BlockSpec(cvec.shape, lambda i: (0, 0)),
            pl.BlockSpec(Wpq.shape, lambda i: (0, 0)),
            pl.BlockSpec(bpq.shape, lambda i: (0, 0)),
        ],
        out_specs=[
            pl.BlockSpec((eblk, deo), lambda i: (i, 0)),
            pl.BlockSpec((2, eblk, deo // 2), lambda i: (0, i, 0)),
            pl.BlockSpec((1, deo), lambda i: (0, 0)),
        ],
        out_shape=[
            jax.ShapeDtypeStruct((E, deo), F32),
            jax.ShapeDtypeStruct((2, E, deo // 2), F32),
            jax.ShapeDtypeStruct((1, deo), F32),
        ],
    )(gd, gs, ea, Wee, cvec, Wpq, bpq)


def _sc_scatter(dst, proj2, z2d, N, E, CH, NB):
    h = proj2.shape[2]
    epw = E // _NS
    nouter = (epw // CH) // NB
    rpt = (N // _NS) // 8 * 8
    rlast = N - (_NS - 1) * rpt

    mesh = plsc.VectorSubcoreMesh(core_axis_name="c", subcore_axis_name="s")
    scratch = (
        [pltpu.VMEM((CH,), jnp.int32) for _ in range(NB)]
        + [pltpu.VMEM((CH, h), F32) for _ in range(NB)]
        + [
            pltpu.VMEM_SHARED((N, h), F32),
            pltpu.SemaphoreType.DMA((2 * NB,)),
        ]
    )

    @functools.partial(
        pl.kernel,
        out_type=jax.ShapeDtypeStruct((2, N, h), F32),
        mesh=mesh,
        scratch_types=scratch,
    )
    def k(dst_h, pj_h, z2_h, sums_h, *sc):
        idx = sc[0:NB]
        pbuf = sc[NB : 2 * NB]
        acc = sc[2 * NB]
        sems = sc[2 * NB + 1]
        cid = lax.axis_index("c")
        sid = lax.axis_index("s")

        def zero_acc():
            @pl.when(sid < _NS - 1)
            def _():
                pltpu.sync_copy(z2_h.at[pl.ds(sid * rpt, rpt)],
                                acc.at[pl.ds(sid * rpt, rpt)])

            @pl.when(sid == _NS - 1)
            def _():
                pltpu.sync_copy(z2_h.at[pl.ds((_NS - 1) * rpt, rlast)],
                                acc.at[pl.ds((_NS - 1) * rpt, rlast)])

        def write_acc(out3d):
            @pl.when(sid < _NS - 1)
            def _():
                pltpu.sync_copy(acc.at[pl.ds(sid * rpt, rpt)],
                                out3d.at[cid, pl.ds(sid * rpt, rpt)])

            @pl.when(sid == _NS - 1)
            def _():
                pltpu.sync_copy(acc.at[pl.ds((_NS - 1) * rpt, rlast)],
                                out3d.at[cid, pl.ds((_NS - 1) * rpt, rlast)])

        zero_acc()
        plsc.subcore_barrier()

        base = sid * epw

        def outer(t, carry):
            cps = []
            for r in range(NB):
                off = base + (t * NB + r) * CH
                cps.append(pltpu.async_copy(
                    dst_h.at[pl.ds(off, CH)], idx[r], sems.at[r]))
                cps.append(pltpu.async_copy(
                    pj_h.at[cid, pl.ds(off, CH)], pbuf[r], sems.at[NB + r]))
            for c in cps:
                c.wait()
            for r in range(NB):
                pltpu.sync_copy(pbuf[r], acc.at[idx[r]], add=True)
            return carry

        lax.fori_loop(0, nouter, outer, 0)
        plsc.subcore_barrier()
        write_acc(sums_h)

    return k(dst, proj2, z2d)


def _sc_counts(dst, z2d, ones_in, N, E, CH, NB):
    h = z2d.shape[1]
    epw_c = E // (2 * _NS)
    nouter_c = (epw_c // CH) // NB
    rpt = (N // _NS) // 8 * 8
    rlast = N - (_NS - 1) * rpt

    mesh = plsc.VectorSubcoreMesh(core_axis_name="c", subcore_axis_name="s")
    scratch = (
        [pltpu.VMEM((CH,), jnp.int32) for _ in range(NB)]
        + [
            pltpu.VMEM((CH, h), F32),
            pltpu.VMEM_SHARED((N, h), F32),
            pltpu.SemaphoreType.DMA((NB,)),
        ]
    )

    @functools.partial(
        pl.kernel,
        out_type=jax.ShapeDtypeStruct((2, N, h), F32),
        mesh=mesh,
        scratch_types=scratch,
    )
    def k(dst_h, z2_h, ones_h, cnt_h, *sc):
        idx = sc[0:NB]
        ones = sc[NB]
        acc = sc[NB + 1]
        sems = sc[NB + 2]
        cid = lax.axis_index("c")
        sid = lax.axis_index("s")

        @pl.when(sid < _NS - 1)
        def _():
            pltpu.sync_copy(z2_h.at[pl.ds(sid * rpt, rpt)],
                            acc.at[pl.ds(sid * rpt, rpt)])

        @pl.when(sid == _NS - 1)
        def _():
            pltpu.sync_copy(z2_h.at[pl.ds((_NS - 1) * rpt, rlast)],
                            acc.at[pl.ds((_NS - 1) * rpt, rlast)])

        pltpu.sync_copy(ones_h, ones)
        plsc.subcore_barrier()

        base_c = cid * (E // 2) + sid * epw_c

        def outer_c(t, carry):
            cps = []
            for r in range(NB):
                off = base_c + (t * NB + r) * CH
                cps.append(pltpu.async_copy(
                    dst_h.at[pl.ds(off, CH)], idx[r], sems.at[r]))
            for c in cps:
                c.wait()
            for r in range(NB):
                pltpu.sync_copy(ones, acc.at[idx[r]], add=True)
            return carry

        lax.fori_loop(0, nouter_c, outer_c, 0)
        plsc.subcore_barrier()

        @pl.when(sid < _NS - 1)
        def _():
            pltpu.sync_copy(acc.at[pl.ds(sid * rpt, rpt)],
                            cnt_h.at[cid, pl.ds(sid * rpt, rpt)])

        @pl.when(sid == _NS - 1)
        def _():
            pltpu.sync_copy(acc.at[pl.ds((_NS - 1) * rpt, rlast)],
                            cnt_h.at[cid, pl.ds((_NS - 1) * rpt, rlast)])

    return k(dst, z2d, ones_in)


def _node_global(sums2, cnt2, Wn, bn2, Wpn, bpn2, pe_sum, u2, Wg, bg2,
                 E, nblk):
    N = sums2.shape[1]
    h = sums2.shape[2]
    d = Wn.shape[0]
    dg = u2.shape[1]
    blk = N // nblk

    def body(s_ref, c_ref, wn_ref, bn_ref, wpn_ref, bpn_ref, pe_ref,
             u_ref, wg_ref, bg_ref, n_ref, u_out_ref, acc):
        s = jnp.concatenate([s_ref[0], s_ref[1]], axis=1)
        cnt = c_ref[0][:, 0:1] + c_ref[1][:, 0:1]
        agg = s / jnp.maximum(cnt, 1.0)
        nb = jnp.maximum(
            jnp.dot(agg, wn_ref[...], preferred_element_type=F32)
            + bn_ref[...],
            0.0,
        )
        n_ref[...] = nb
        part = jnp.sum(
            jnp.maximum(
                jnp.dot(nb, wpn_ref[...], preferred_element_type=F32)
                + bpn_ref[...],
                0.0,
            ),
            axis=0,
            keepdims=True,
        )

        @pl.when(pl.program_id(0) == 0)
        def _():
            acc[...] = jnp.zeros_like(acc)

        acc[...] += part

        @pl.when(pl.program_id(0) == pl.num_programs(0) - 1)
        def _():
            ge = pe_ref[...] / float(E)
            gn = acc[...] / float(N)
            g = (
                jnp.dot(ge, wg_ref[: 2 * h, :], preferred_element_type=F32)
                + jnp.dot(gn, wg_ref[2 * h : 2 * h + d, :],
                          preferred_element_type=F32)
                + jnp.dot(u_ref[...], wg_ref[2 * h + d :, :],
                          preferred_element_type=F32)
                + bg_ref[...]
            )
            u_out_ref[...] = jnp.maximum(g, 0.0)

    return pl.pallas_call(
        body,
        grid=(nblk,),
        in_specs=[
            pl.BlockSpec((2, blk, h), lambda i: (0, i, 0)),
            pl.BlockSpec((2, blk, h), lambda i: (0, i, 0)),
            pl.BlockSpec(Wn.shape, lambda i: (0, 0)),
            pl.BlockSpec(bn2.shape, lambda i: (0, 0)),
            pl.BlockSpec(Wpn.shape, lambda i: (0, 0)),
            pl.BlockSpec(bpn2.shape, lambda i: (0, 0)),
            pl.BlockSpec(pe_sum.shape, lambda i: (0, 0)),
            pl.BlockSpec(u2.shape, lambda i: (0, 0)),
            pl.BlockSpec(Wg.shape, lambda i: (0, 0)),
            pl.BlockSpec(bg2.shape, lambda i: (0, 0)),
        ],
        out_specs=[
            pl.BlockSpec((blk, d), lambda i: (i, 0)),
            pl.BlockSpec((1, dg), lambda i: (0, 0)),
        ],
        out_shape=[
            jax.ShapeDtypeStruct((N, d), F32),
            jax.ShapeDtypeStruct((1, dg), F32),
        ],
        scratch_shapes=[pltpu.VMEM((1, d), F32)],
    )(sums2, cnt2, Wn, bn2, Wpn, bpn2, pe_sum, u2, Wg, bg2)


def kernel(x, edge_index, edge_attr, u, We, be, Wp, bp, Wn, bn,
           Wpe, bpe, Wpn, bpn, Wg, bg):
    N, D = x.shape
    E = edge_attr.shape[0]
    DE = edge_attr.shape[1]
    DG = u.shape[0]
    DEO = We.shape[1]

    src = edge_index[0]
    dst = edge_index[1]

    Wee = We[:DE]
    Wrs = jnp.concatenate([We[DE : DE + D], We[DE + D : DE + 2 * D]], axis=1)
    Weu = We[DE + 2 * D :]
    u2 = u.reshape(1, DG)
    be2 = be.reshape(1, DEO)
    Wpq = jnp.concatenate([Wp, Wpe], axis=1)
    bpq = jnp.concatenate([bp, bpe]).reshape(1, 2 * DEO)

    xr, xs, cvec = _precompute(x, Wrs, u2, Weu, be2, nblk=10)
    gd, gs = _sc_gather(dst, src, xr, xs, E, DEO, CH=40, NB=5)
    z2d = jnp.zeros((N, DEO // 2), F32)
    ones_in = jnp.ones((40, DEO // 2), F32)
    cnt2 = _sc_counts(dst, z2d, ones_in, N, E, CH=40, NB=5)
    e_new, proj2, pe_sum = _edge_block(gd, gs, edge_attr, Wee, cvec,
                                       Wpq, bpq, eblk=2000)
    sums2 = _sc_scatter(dst, proj2, z2d, N, E, CH=40, NB=5)
    n_new, u_new2 = _node_global(sums2, cnt2, Wn, bn.reshape(1, D),
                                 Wpn, bpn.reshape(1, D), pe_sum, u2,
                                 Wg, bg.reshape(1, DG), E, nblk=10)
    return e_new, n_new, u_new2.reshape(DG)


# speedup vs baseline: 3.2335x; 1.0454x over previous
import functools

import jax
import jax.numpy as jnp
from jax import lax
from jax.experimental import pallas as pl
from jax.experimental.pallas import tpu as pltpu
from jax.experimental.pallas import tpu_sc as plsc

F32 = jnp.float32

_NC = 2
_NS = 16
_NW = _NC * _NS


def _precompute(x, Wrs, u2, Weu, be2, nblk):
    n, d = x.shape
    deo = Weu.shape[1]
    blk = n // nblk

    def body(x_ref, w_ref, u_ref, weu_ref, be_ref, xr_ref, xs_ref, cv_ref):
        t = jnp.dot(x_ref[...], w_ref[...], preferred_element_type=F32)
        xr_ref[...] = t[:, :deo]
        xs_ref[...] = t[:, deo:]

        @pl.when(pl.program_id(0) == 0)
        def _():
            cv_ref[...] = (
                jnp.dot(u_ref[...], weu_ref[...], preferred_element_type=F32)
                + be_ref[...]
            )

    return pl.pallas_call(
        body,
        grid=(nblk,),
        in_specs=[
            pl.BlockSpec((blk, d), lambda i: (i, 0)),
            pl.BlockSpec((d, 2 * deo), lambda i: (0, 0)),
            pl.BlockSpec(u2.shape, lambda i: (0, 0)),
            pl.BlockSpec(Weu.shape, lambda i: (0, 0)),
            pl.BlockSpec(be2.shape, lambda i: (0, 0)),
        ],
        out_specs=[
            pl.BlockSpec((blk, deo), lambda i: (i, 0)),
            pl.BlockSpec((blk, deo), lambda i: (i, 0)),
            pl.BlockSpec((1, deo), lambda i: (0, 0)),
        ],
        out_shape=[
            jax.ShapeDtypeStruct((n, deo), F32),
            jax.ShapeDtypeStruct((n, deo), F32),
            jax.ShapeDtypeStruct((1, deo), F32),
        ],
    )(x, Wrs, u2, Weu, be2)


def _sc_gather(dst, src, xr, xs, E, DEO, CH, NB):
    epw = E // _NW
    m = epw // CH
    nouter = m // NB

    mesh = plsc.VectorSubcoreMesh(core_axis_name="c", subcore_axis_name="s")
    scratch = (
        [pltpu.VMEM((CH,), jnp.int32) for _ in range(2 * NB)]
        + [pltpu.VMEM((CH, DEO), F32) for _ in range(2 * NB)]
        + [pltpu.SemaphoreType.DMA((3 * NB,))]
    )

    @functools.partial(
        pl.kernel,
        out_type=(
            jax.ShapeDtypeStruct((E, DEO), F32),
            jax.ShapeDtypeStruct((E, DEO), F32),
        ),
        mesh=mesh,
        scratch_types=scratch,
    )
    def k(dst_h, src_h, xr_h, xs_h, gd_h, gs_h, *sc):
        idx_d = sc[0:NB]
        idx_s = sc[NB : 2 * NB]
        row_d = sc[2 * NB : 3 * NB]
        row_s = sc[3 * NB : 4 * NB]
        sems = sc[4 * NB]
        wid = lax.axis_index("s") * _NC + lax.axis_index("c")
        base = wid * epw

        def wb_addr(tt, r):
            off = base + (tt * NB + r) * CH
            return gd_h.at[pl.ds(off, CH)], gs_h.at[pl.ds(off, CH)]

        for r in range(NB):
            off = base + r * CH
            pltpu.async_copy(dst_h.at[pl.ds(off, CH)], idx_d[r], sems.at[r])
            pltpu.async_copy(src_h.at[pl.ds(off, CH)], idx_s[r], sems.at[r])

        def outer(t, carry):
            @pl.when(t > 0)
            def _():
                for r in range(NB):
                    gd_s, gs_s = wb_addr(t - 1, r)
                    pltpu.make_async_copy(row_d[r], gd_s,
                                          sems.at[2 * NB + r]).wait()
                    pltpu.make_async_copy(row_s[r], gs_s,
                                          sems.at[2 * NB + r]).wait()
            for r in range(NB):
                off = base + (t * NB + r) * CH
                pltpu.make_async_copy(dst_h.at[pl.ds(off, CH)], idx_d[r],
                                      sems.at[r]).wait()
                pltpu.make_async_copy(src_h.at[pl.ds(off, CH)], idx_s[r],
                                      sems.at[r]).wait()
                pltpu.async_copy(xr_h.at[idx_d[r]], row_d[r], sems.at[NB + r])
                pltpu.async_copy(xs_h.at[idx_s[r]], row_s[r], sems.at[NB + r])
            for r in range(NB):
                pltpu.make_async_copy(xr_h.at[idx_d[r]], row_d[r],
                                      sems.at[NB + r]).wait()
                pltpu.make_async_copy(xs_h.at[idx_s[r]], row_s[r],
                                      sems.at[NB + r]).wait()

                @pl.when(t + 1 < nouter)
                def _():
                    off_n = base + ((t + 1) * NB + r) * CH
                    pltpu.async_copy(dst_h.at[pl.ds(off_n, CH)], idx_d[r],
                                     sems.at[r])
                    pltpu.async_copy(src_h.at[pl.ds(off_n, CH)], idx_s[r],
                                     sems.at[r])

                gd_s, gs_s = wb_addr(t, r)
                pltpu.async_copy(row_d[r], gd_s, sems.at[2 * NB + r])
                pltpu.async_copy(row_s[r], gs_s, sems.at[2 * NB + r])
            return carry

        lax.fori_loop(0, nouter, outer, 0)
        for r in range(NB):
            gd_s, gs_s = wb_addr(nouter - 1, r)
            pltpu.make_async_copy(row_d[r], gd_s, sems.at[2 * NB + r]).wait()
            pltpu.make_async_copy(row_s[r], gs_s, sems.at[2 * NB + r]).wait()

    return k(dst, src, xr, xs)


def _edge_block(gd, gs, ea, Wee, cvec, Wpq, bpq, eblk):
    E, deo = gd.shape
    de = ea.shape[1]
    dq = Wpq.shape[1]
    nblk = E // eblk

    def body(gd_ref, gs_ref, ea_ref, wee_ref, cv_ref, wpq_ref, bpq_ref,
             en_ref, pj_ref, pe_ref):
        pre = (
            gd_ref[...] + gs_ref[...]
            + jnp.dot(ea_ref[...], wee_ref[...], preferred_element_type=F32)
            + cv_ref[...]
        )
        e_new = jnp.maximum(pre, 0.0)
        en_ref[...] = e_new
        pq = jnp.maximum(
            jnp.dot(e_new.astype(jnp.bfloat16),
                    wpq_ref[...].astype(jnp.bfloat16),
                    preferred_element_type=F32)
            + bpq_ref[...],
            0.0,
        )
        h = deo // 2
        pj_ref[...] = jnp.stack([pq[:, :h], pq[:, h : 2 * h]], axis=0)
        part = jnp.sum(pq[:, deo:], axis=0, keepdims=True)

        @pl.when(pl.program_id(0) == 0)
        def _():
            pe_ref[...] = jnp.zeros_like(pe_ref)

        pe_ref[...] += part

    return pl.pallas_call(
        body,
        grid=(nblk,),
        in_specs=[
            pl.BlockSpec((eblk, deo), lambda i: (i, 0)),
            pl.BlockSpec((eblk, deo), lambda i: (i, 0)),
            pl.BlockSpec((eblk, de), lambda i: (i, 0)),
            pl.BlockSpec(Wee.shape, lambda i: (0, 0)),
            pl.BlockSpec(cvec.shape, lambda i: (0, 0)),
            pl.BlockSpec(Wpq.shape, lambda i: (0, 0)),
            pl.BlockSpec(bpq.shape, lambda i: (0, 0)),
        ],
        out_specs=[
            pl.BlockSpec((eblk, deo), lambda i: (i, 0)),
            pl.BlockSpec((2, eblk, deo // 2), lambda i: (0, i, 0)),
            pl.BlockSpec((1, deo), lambda i: (0, 0)),
        ],
        out_shape=[
            jax.ShapeDtypeStruct((E, deo), F32),
            jax.ShapeDtypeStruct((2, E, deo // 2), F32),
            jax.ShapeDtypeStruct((1, deo), F32),
        ],
    )(gd, gs, ea, Wee, cvec, Wpq, bpq)


def _sc_scatter(dst, proj2, z2d, N, E, CH, NB):
    h = proj2.shape[2]
    epw = E // _NS
    nouter = (epw // CH) // NB
    rpt = (N // _NS) // 8 * 8
    rlast = N - (_NS - 1) * rpt

    mesh = plsc.VectorSubcoreMesh(core_axis_name="c", subcore_axis_name="s")
    scratch = (
        [pltpu.VMEM((CH,), jnp.int32) for _ in range(NB)]
        + [pltpu.VMEM((CH, h), F32) for _ in range(NB)]
        + [
            pltpu.VMEM_SHARED((N, h), F32),
            pltpu.SemaphoreType.DMA((3 * NB,)),
        ]
    )

    @functools.partial(
        pl.kernel,
        out_type=jax.ShapeDtypeStruct((2, N, h), F32),
        mesh=mesh,
        scratch_types=scratch,
    )
    def k(dst_h, pj_h, z2_h, sums_h, *sc):
        idx = sc[0:NB]
        pbuf = sc[NB : 2 * NB]
        acc = sc[2 * NB]
        sems = sc[2 * NB + 1]
        cid = lax.axis_index("c")
        sid = lax.axis_index("s")

        def zero_acc():
            @pl.when(sid < _NS - 1)
            def _():
                pltpu.sync_copy(z2_h.at[pl.ds(sid * rpt, rpt)],
                                acc.at[pl.ds(sid * rpt, rpt)])

            @pl.when(sid == _NS - 1)
            def _():
                pltpu.sync_copy(z2_h.at[pl.ds((_NS - 1) * rpt, rlast)],
                                acc.at[pl.ds((_NS - 1) * rpt, rlast)])

        def write_acc(out3d):
            @pl.when(sid < _NS - 1)
            def _():
                pltpu.sync_copy(acc.at[pl.ds(sid * rpt, rpt)],
                                out3d.at[cid, pl.ds(sid * rpt, rpt)])

            @pl.when(sid == _NS - 1)
            def _():
                pltpu.sync_copy(acc.at[pl.ds((_NS - 1) * rpt, rlast)],
                                out3d.at[cid, pl.ds((_NS - 1) * rpt, rlast)])

        zero_acc()
        plsc.subcore_barrier()

        base = sid * epw

        def outer(t, carry):
            cps = []
            for r in range(NB):
                off = base + (t * NB + r) * CH
                cps.append(pltpu.async_copy(
                    dst_h.at[pl.ds(off, CH)], idx[r], sems.at[r]))
                cps.append(pltpu.async_copy(
                    pj_h.at[cid, pl.ds(off, CH)], pbuf[r], sems.at[NB + r]))
            scats = []
            for r in range(NB):
                cps[2 * r].wait()
                cps[2 * r + 1].wait()
                scats.append(pltpu.async_copy(
                    pbuf[r], acc.at[idx[r]], sems.at[2 * NB + r], add=True))
            for c in scats:
                c.wait()
            return carry

        lax.fori_loop(0, nouter, outer, 0)
        plsc.subcore_barrier()
        write_acc(sums_h)

    return k(dst, proj2, z2d)


def _sc_counts(dst, z2d, ones_in, N, E, CH, NB):
    h = z2d.shape[1]
    epw_c = E // (2 * _NS)
    nouter_c = (epw_c // CH) // NB
    rpt = (N // _NS) // 8 * 8
    rlast = N - (_NS - 1) * rpt

    mesh = plsc.VectorSubcoreMesh(core_axis_name="c", subcore_axis_name="s")
    scratch = (
        [pltpu.VMEM((CH,), jnp.int32) for _ in range(NB)]
        + [
            pltpu.VMEM((CH, h), F32),
            pltpu.VMEM_SHARED((N, h), F32),
            pltpu.SemaphoreType.DMA((NB,)),
        ]
    )

    @functools.partial(
        pl.kernel,
        out_type=jax.ShapeDtypeStruct((2, N, h), F32),
        mesh=mesh,
        scratch_types=scratch,
    )
    def k(dst_h, z2_h, ones_h, cnt_h, *sc):
        idx = sc[0:NB]
        ones = sc[NB]
        acc = sc[NB + 1]
        sems = sc[NB + 2]
        cid = lax.axis_index("c")
        sid = lax.axis_index("s")

        @pl.when(sid < _NS - 1)
        def _():
            pltpu.sync_copy(z2_h.at[pl.ds(sid * rpt, rpt)],
                            acc.at[pl.ds(sid * rpt, rpt)])

        @pl.when(sid == _NS - 1)
        def _():
            pltpu.sync_copy(z2_h.at[pl.ds((_NS - 1) * rpt, rlast)],
                            acc.at[pl.ds((_NS - 1) * rpt, rlast)])

        pltpu.sync_copy(ones_h, ones)
        plsc.subcore_barrier()

        base_c = cid * (E // 2) + sid * epw_c

        def outer_c(t, carry):
            cps = []
            for r in range(NB):
                off = base_c + (t * NB + r) * CH
                cps.append(pltpu.async_copy(
                    dst_h.at[pl.ds(off, CH)], idx[r], sems.at[r]))
            for c in cps:
                c.wait()
            for r in range(NB):
                pltpu.sync_copy(ones, acc.at[idx[r]], add=True)
            return carry

        lax.fori_loop(0, nouter_c, outer_c, 0)
        plsc.subcore_barrier()

        @pl.when(sid < _NS - 1)
        def _():
            pltpu.sync_copy(acc.at[pl.ds(sid * rpt, rpt)],
                            cnt_h.at[cid, pl.ds(sid * rpt, rpt)])

        @pl.when(sid == _NS - 1)
        def _():
            pltpu.sync_copy(acc.at[pl.ds((_NS - 1) * rpt, rlast)],
                            cnt_h.at[cid, pl.ds((_NS - 1) * rpt, rlast)])

    return k(dst, z2d, ones_in)


def _node_global(sums2, cnt2, Wn, bn2, Wpn, bpn2, pe_sum, u2, Wg, bg2,
                 E, nblk):
    N = sums2.shape[1]
    h = sums2.shape[2]
    d = Wn.shape[0]
    dg = u2.shape[1]
    blk = N // nblk

    def body(s_ref, c_ref, wn_ref, bn_ref, wpn_ref, bpn_ref, pe_ref,
             u_ref, wg_ref, bg_ref, n_ref, u_out_ref, acc):
        s = jnp.concatenate([s_ref[0], s_ref[1]], axis=1)
        cnt = c_ref[0][:, 0:1] + c_ref[1][:, 0:1]
        agg = s / jnp.maximum(cnt, 1.0)
        nb = jnp.maximum(
            jnp.dot(agg, wn_ref[...], preferred_element_type=F32)
            + bn_ref[...],
            0.0,
        )
        n_ref[...] = nb
        part = jnp.sum(
            jnp.maximum(
                jnp.dot(nb, wpn_ref[...], preferred_element_type=F32)
                + bpn_ref[...],
                0.0,
            ),
            axis=0,
            keepdims=True,
        )

        @pl.when(pl.program_id(0) == 0)
        def _():
            acc[...] = jnp.zeros_like(acc)

        acc[...] += part

        @pl.when(pl.program_id(0) == pl.num_programs(0) - 1)
        def _():
            ge = pe_ref[...] / float(E)
            gn = acc[...] / float(N)
            g = (
                jnp.dot(ge, wg_ref[: 2 * h, :], preferred_element_type=F32)
                + jnp.dot(gn, wg_ref[2 * h : 2 * h + d, :],
                          preferred_element_type=F32)
                + jnp.dot(u_ref[...], wg_ref[2 * h + d :, :],
                          preferred_element_type=F32)
                + bg_ref[...]
            )
            u_out_ref[...] = jnp.maximum(g, 0.0)

    return pl.pallas_call(
        body,
        grid=(nblk,),
        in_specs=[
            pl.BlockSpec((2, blk, h), lambda i: (0, i, 0)),
            pl.BlockSpec((2, blk, h), lambda i: (0, i, 0)),
            pl.BlockSpec(Wn.shape, lambda i: (0, 0)),
            pl.BlockSpec(bn2.shape, lambda i: (0, 0)),
            pl.BlockSpec(Wpn.shape, lambda i: (0, 0)),
            pl.BlockSpec(bpn2.shape, lambda i: (0, 0)),
            pl.BlockSpec(pe_sum.shape, lambda i: (0, 0)),
            pl.BlockSpec(u2.shape, lambda i: (0, 0)),
            pl.BlockSpec(Wg.shape, lambda i: (0, 0)),
            pl.BlockSpec(bg2.shape, lambda i: (0, 0)),
        ],
        out_specs=[
            pl.BlockSpec((blk, d), lambda i: (i, 0)),
            pl.BlockSpec((1, dg), lambda i: (0, 0)),
        ],
        out_shape=[
            jax.ShapeDtypeStruct((N, d), F32),
            jax.ShapeDtypeStruct((1, dg), F32),
        ],
        scratch_shapes=[pltpu.VMEM((1, d), F32)],
    )(sums2, cnt2, Wn, bn2, Wpn, bpn2, pe_sum, u2, Wg, bg2)


def kernel(x, edge_index, edge_attr, u, We, be, Wp, bp, Wn, bn,
           Wpe, bpe, Wpn, bpn, Wg, bg):
    N, D = x.shape
    E = edge_attr.shape[0]
    DE = edge_attr.shape[1]
    DG = u.shape[0]
    DEO = We.shape[1]

    src = edge_index[0]
    dst = edge_index[1]

    Wee = We[:DE]
    Wrs = jnp.concatenate([We[DE : DE + D], We[DE + D : DE + 2 * D]], axis=1)
    Weu = We[DE + 2 * D :]
    u2 = u.reshape(1, DG)
    be2 = be.reshape(1, DEO)
    Wpq = jnp.concatenate([Wp, Wpe], axis=1)
    bpq = jnp.concatenate([bp, bpe]).reshape(1, 2 * DEO)

    xr, xs, cvec = _precompute(x, Wrs, u2, Weu, be2, nblk=10)
    gd, gs = _sc_gather(dst, src, xr, xs, E, DEO, CH=40, NB=5)
    z2d = jnp.zeros((N, DEO // 2), F32)
    ones_in = jnp.ones((40, DEO // 2), F32)
    cnt2 = _sc_counts(dst, z2d, ones_in, N, E, CH=40, NB=5)
    e_new, proj2, pe_sum = _edge_block(gd, gs, edge_attr, Wee, cvec,
                                       Wpq, bpq, eblk=2000)
    sums2 = _sc_scatter(dst, proj2, z2d, N, E, CH=40, NB=5)
    n_new, u_new2 = _node_global(sums2, cnt2, Wn, bn.reshape(1, D),
                                 Wpn, bpn.reshape(1, D), pe_sum, u2,
                                 Wg, bg.reshape(1, DG), E, nblk=10)
    return e_new, n_new, u_new2.reshape(DG)
